# Initial kernel scaffold; baseline (speedup 1.0000x reference)
import jax
import jax.numpy as jnp
from jax.experimental import pallas as pl


def kernel(x, edge_index, W_first, b_first, W1, b1, W2, b2, W_prep, b_prep, W_cls, b_cls):
    raise NotImplementedError("write your pallas kernel here")


# speedup vs baseline: 5.8882x; 5.8882x over previous
import functools

import jax
import jax.numpy as jnp
from jax import lax
from jax.experimental import pallas as pl
from jax.experimental.pallas import tpu as pltpu
from jax.experimental.pallas import tpu_sc as plsc

N = 100000
E = 1600000
HP = 48
Q = 26624
AGG_ROWS = 4 * Q
SP_ROWS = Q + 2048
DEG_ROWS = 102400
E_PAD = 1605632
EC = E_PAD // 128
N_TC = 100352
R_TC = 512
G_TC = N_TC // R_TC

_f32 = jnp.float32
_i32 = jnp.int32


def _sc_prep_body(dst_ref, ones_ref, zeros_ref, deg_out, dstloc_out,
                  dstv, dlv, o2, z16, deg_sh):
    c = lax.axis_index("c")
    s = lax.axis_index("s")
    wid = c * 16 + s

    pltpu.sync_copy(zeros_ref, z16)
    pltpu.sync_copy(ones_ref, o2)

    zb = s * (DEG_ROWS // 16)
    def zero_step(k, _):
        pltpu.sync_copy(z16, deg_sh.at[pl.ds(zb + k * 128, 128), :])
        return _
    lax.fori_loop(0, DEG_ROWS // 16 // 128, zero_step, None)
    plsc.subcore_barrier()

    rows_per_tile = EC // 32
    def chunk(m, _):
        r0 = wid * rows_per_tile + m * 8
        pltpu.sync_copy(dst_ref.at[pl.ds(r0, 8), :], dstv)
        for j in range(8):
            for t in range(8):
                d = dstv[j, pl.ds(t * 16, 16)]
                for q in range(4):
                    base = q * Q
                    in_r = (d >= base) & (d < base + Q)
                    loc = jnp.where(in_r, d - base, Q)
                    dlv[q, j, pl.ds(t * 16, 16)] = loc
        for j in range(8):
            pltpu.sync_copy(o2, deg_sh.at[dstv.at[j]], add=True)
        for q in range(4):
            pltpu.sync_copy(dlv.at[q], dstloc_out.at[q, pl.ds(r0, 8), :])
        return _
    lax.fori_loop(0, rows_per_tile // 8, chunk, None)
    plsc.subcore_barrier()

    def wb_step(k, _):
        pltpu.sync_copy(deg_sh.at[pl.ds(zb + k * 128, 128), :],
                        deg_out.at[c, pl.ds(zb + k * 128, 128), :])
        return _
    lax.fori_loop(0, DEG_ROWS // 16 // 128, wb_step, None)


@jax.jit
def _sc_prep(dstp, ones16, zeros16):
    mesh = plsc.VectorSubcoreMesh(core_axis_name="c", subcore_axis_name="s")
    return pl.kernel(
        _sc_prep_body,
        out_type=[jax.ShapeDtypeStruct((2, DEG_ROWS, 8), _f32),
                  jax.ShapeDtypeStruct((4, EC, 128), _i32)],
        mesh=mesh,
        compiler_params=pltpu.CompilerParams(use_tc_tiling_on_sc=False),
        scratch_types=[
            pltpu.VMEM((8, 128), _i32),
            pltpu.VMEM((4, 8, 128), _i32),
            pltpu.VMEM((128, 8), _f32),
            pltpu.VMEM((128, 8), _f32),
            pltpu.VMEM_SHARED((DEG_ROWS, 8), _f32),
        ],
    )(dstp, ones16, zeros16)


def _sc_agg_body(g_ref, src_ref, dstloc_ref, zeros_ref, agg_out,
                 sv, dv, rows, z48, agg_sh, sem):
    c = lax.axis_index("c")
    s = lax.axis_index("s")
    rows_per_tile = EC // 16

    pltpu.sync_copy(zeros_ref, z48)

    for p in range(2):
        q = 2 * c + p
        zb = s * (SP_ROWS // 16)
        def zero_step(k, _):
            pltpu.sync_copy(z48, agg_sh.at[pl.ds(zb + k * 128, 128), :])
            return _
        lax.fori_loop(0, SP_ROWS // 16 // 128, zero_step, None)
        plsc.subcore_barrier()

        def chunk(m, _):
            r0 = s * rows_per_tile + m * 4
            pltpu.sync_copy(src_ref.at[pl.ds(r0, 4), :], sv)
            pltpu.sync_copy(dstloc_ref.at[q, pl.ds(r0, 4), :], dv)
            descs = []
            for j in range(4):
                descs.append(pltpu.async_copy(
                    g_ref.at[sv.at[j]], rows.at[j], sem))
            for d in descs:
                d.wait()
            for j in range(4):
                pltpu.sync_copy(rows.at[j], agg_sh.at[dv.at[j]], add=True)
            return _
        lax.fori_loop(0, rows_per_tile // 4, chunk, None)
        plsc.subcore_barrier()

        wb = s * (Q // 16)
        def wb_step(k, _):
            pltpu.sync_copy(agg_sh.at[pl.ds(wb + k * 128, 128), :],
                            agg_out.at[pl.ds(q * Q + wb + k * 128, 128), :])
            return _
        lax.fori_loop(0, Q // 16 // 128, wb_step, None)
        plsc.subcore_barrier()


@jax.jit
def _sc_agg(g, srcp, dstloc, zeros48):
    mesh = plsc.VectorSubcoreMesh(core_axis_name="c", subcore_axis_name="s")
    return pl.kernel(
        _sc_agg_body,
        out_type=jax.ShapeDtypeStruct((AGG_ROWS, HP), _f32),
        mesh=mesh,
        compiler_params=pltpu.CompilerParams(use_tc_tiling_on_sc=False),
        scratch_types=[
            pltpu.VMEM((4, 128), _i32),
            pltpu.VMEM((4, 128), _i32),
            pltpu.VMEM((4, 128, HP), _f32),
            pltpu.VMEM((128, HP), _f32),
            pltpu.VMEM_SHARED((SP_ROWS, HP), _f32),
            pltpu.SemaphoreType.DMA,
        ],
    )(g, srcp, dstloc, zeros48)


def _tc_a_body(x_ref, wf_ref, bf_ref, w1_ref, dega_ref, degb_ref,
               g1_ref, dinv_ref):
    h0 = jnp.dot(x_ref[...], wf_ref[...],
                 preferred_element_type=_f32) + bf_ref[...]
    deg = dega_ref[...][:, :1] + degb_ref[...][:, :1] + 1.0
    dinv = lax.rsqrt(jnp.maximum(deg, 1.0))
    hw = jnp.dot(h0, w1_ref[...], preferred_element_type=_f32)
    g1_ref[...] = hw * dinv
    dinv_ref[...] = jnp.broadcast_to(dinv, (R_TC, 16))


@jax.jit
def _tc_a(xp, wf, bf, w1, dega, degb):
    return pl.pallas_call(
        _tc_a_body,
        grid=(G_TC,),
        in_specs=[pl.BlockSpec((R_TC, 128), lambda i: (i, 0)),
                  pl.BlockSpec((128, HP), lambda i: (0, 0)),
                  pl.BlockSpec((1, HP), lambda i: (0, 0)),
                  pl.BlockSpec((HP, HP), lambda i: (0, 0)),
                  pl.BlockSpec((R_TC, 8), lambda i: (i, 0)),
                  pl.BlockSpec((R_TC, 8), lambda i: (i, 0))],
        out_specs=[pl.BlockSpec((R_TC, HP), lambda i: (i, 0)),
                   pl.BlockSpec((R_TC, 16), lambda i: (i, 0))],
        out_shape=[jax.ShapeDtypeStruct((N_TC, HP), _f32),
                   jax.ShapeDtypeStruct((N_TC, 16), _f32)],
    )(xp, wf, bf, w1, dega, degb)


def _tc_b_body(agg_ref, g_ref, dinv_ref, b_ref, w_ref, gout_ref):
    dinv = dinv_ref[...][:, :1]
    h = jnp.tanh(dinv * (agg_ref[...] + g_ref[...]) + b_ref[...])
    gout_ref[...] = jnp.dot(h, w_ref[...], preferred_element_type=_f32) * dinv


@jax.jit
def _tc_b(agg, g, dinv16, b, w):
    return pl.pallas_call(
        _tc_b_body,
        grid=(G_TC,),
        in_specs=[pl.BlockSpec((R_TC, HP), lambda i: (i, 0)),
                  pl.BlockSpec((R_TC, HP), lambda i: (i, 0)),
                  pl.BlockSpec((R_TC, 16), lambda i: (i, 0)),
                  pl.BlockSpec((1, HP), lambda i: (0, 0)),
                  pl.BlockSpec((HP, HP), lambda i: (0, 0))],
        out_specs=pl.BlockSpec((R_TC, HP), lambda i: (i, 0)),
        out_shape=jax.ShapeDtypeStruct((N_TC, HP), _f32),
    )(agg, g, dinv16, b, w)


def _tc_c_body(agg_ref, g_ref, dinv_ref, b_ref, wp_ref, bp_ref,
               wc_ref, bc_ref, out_ref, hp_ref):
    dinv = dinv_ref[...][:, :1]
    h2 = jnp.tanh(dinv * (agg_ref[...] + g_ref[...]) + b_ref[...])
    hp = jnp.tanh(jnp.dot(h2, wp_ref[...],
                          preferred_element_type=_f32) + bp_ref[...])
    out = jnp.dot(hp, wc_ref[...], preferred_element_type=_f32) + bc_ref[...]
    out_ref[...] = out
    hp_ref[...] = hp


@jax.jit
def _tc_c(agg, g, dinv16, b, wp, bp, wc, bc):
    return pl.pallas_call(
        _tc_c_body,
        grid=(G_TC,),
        in_specs=[pl.BlockSpec((R_TC, HP), lambda i: (i, 0)),
                  pl.BlockSpec((R_TC, HP), lambda i: (i, 0)),
                  pl.BlockSpec((R_TC, 16), lambda i: (i, 0)),
                  pl.BlockSpec((1, HP), lambda i: (0, 0)),
                  pl.BlockSpec((HP, 8), lambda i: (0, 0)),
                  pl.BlockSpec((1, 8), lambda i: (0, 0)),
                  pl.BlockSpec((8, 8), lambda i: (0, 0)),
                  pl.BlockSpec((1, 8), lambda i: (0, 0))],
        out_specs=[pl.BlockSpec((R_TC, 8), lambda i: (i, 0)),
                   pl.BlockSpec((R_TC, 8), lambda i: (i, 0))],
        out_shape=[jax.ShapeDtypeStruct((N_TC, 8), _f32),
                   jax.ShapeDtypeStruct((N_TC, 8), _f32)],
    )(agg, g, dinv16, b, wp, bp, wc, bc)


def kernel(x, edge_index, W_first, b_first, W1, b1, W2, b2,
           W_prep, b_prep, W_cls, b_cls):
    src = edge_index[0]
    dst = edge_index[1]
    srcp = jnp.concatenate(
        [src, jnp.zeros((E_PAD - E,), _i32)]).reshape(EC, 128)
    dstp = jnp.concatenate(
        [dst, jnp.full((E_PAD - E,), N, _i32)]).reshape(EC, 128)
    xp = jnp.pad(x, ((0, N_TC - N), (0, 0)))

    wf = jnp.pad(W_first, ((0, 0), (0, HP - 34)))
    bf = jnp.pad(b_first, (0, HP - 34)).reshape(1, HP)
    w1 = jnp.pad(W1, ((0, HP - 34), (0, HP - 34)))
    b1p = jnp.pad(b1, (0, HP - 34)).reshape(1, HP)
    w2 = jnp.pad(W2, ((0, HP - 34), (0, HP - 34)))
    b2p = jnp.pad(b2, (0, HP - 34)).reshape(1, HP)
    wp = jnp.pad(W_prep, ((0, HP - 34), (0, 6)))
    bpp = jnp.pad(b_prep, (0, 6)).reshape(1, 8)
    wc = jnp.pad(W_cls, ((0, 6), (0, 4)))
    bcp = jnp.pad(b_cls, (0, 4)).reshape(1, 8)

    ones16 = jnp.ones((128, 8), _f32)
    zeros16 = jnp.zeros((128, 8), _f32)
    zeros48 = jnp.zeros((128, HP), _f32)

    deg2, dstloc = _sc_prep(dstp, ones16, zeros16)
    dega = deg2[0, :N_TC]
    degb = deg2[1, :N_TC]

    g1, dinv16 = _tc_a(xp, wf, bf, w1, dega, degb)
    agg1 = _sc_agg(g1, srcp, dstloc, zeros48)
    g2 = _tc_b(agg1, g1, dinv16, b1p, w2)
    agg2 = _sc_agg(g2, srcp, dstloc, zeros48)
    out8, hp8 = _tc_c(agg2, g2, dinv16, b2p, wp, bpp, wc, bcp)

    return (out8[:N, :4], hp8[:N, :2])
